# baseline (speedup 1.0000x reference)
import functools

import jax
import jax.numpy as jnp
from jax import lax
from jax.experimental import pallas as pl
from jax.experimental.pallas import tpu as pltpu
from jax.experimental.pallas import tpu_sc as plsc

_R = 8
_CAP_E = 8192
_WIN = 512
_CAP_B = _CAP_E + _WIN


@functools.cache
def _build(n: int, v: int, d: int):
    info = plsc.get_sparse_core_info()
    nc, ns = info.num_cores, info.num_subcores
    nw = nc * ns
    assert v % nw == 0 and (v // nw) % _R == 0 and n % 16 == 0
    span = v // nw
    n_sub = span // _R
    n_idx_chunks = 16
    chunk_len = n // n_idx_chunks
    assert chunk_len % 16 == 0

    mesh = plsc.VectorSubcoreMesh(core_axis_name="c", subcore_axis_name="s")

    @functools.partial(
        pl.kernel,
        out_type=jax.ShapeDtypeStruct((n, d), jnp.float32),
        mesh=mesh,
        compiler_params=pltpu.CompilerParams(needs_layout_passes=False),
        scratch_types=[
            pltpu.VMEM((chunk_len,), jnp.int32),
            pltpu.VMEM((_CAP_B,), jnp.int32),
            pltpu.VMEM((_CAP_B,), jnp.int32),
            pltpu.VMEM((_CAP_B,), jnp.int32),
            pltpu.VMEM((_CAP_B,), jnp.int32),
            pltpu.VMEM((_R, d), jnp.float32),
            pltpu.VMEM((16,), jnp.int32),
            pltpu.SemaphoreType.DMA,
            pltpu.SemaphoreType.DMA,
        ],
    )
    def body(idx_hbm, table_hbm, out_hbm, idxbuf, e_pos, e_key,
             g_pos, g_key, rows, ovf_key, sem_row, sem_out):
        wid = lax.axis_index("s") * nc + lax.axis_index("c")
        lo = wid * span
        hi = lo + span
        lane = lax.iota(jnp.int32, 16)

        def lane_at(vec, j):
            return jnp.sum(jnp.where(lane == j, vec, 0))

        def drain_outs(cnt):
            def w(_, c):
                pltpu.make_async_copy(
                    rows.at[0], out_hbm.at[0], sem_out).wait()
                return c
            lax.fori_loop(0, cnt, w, 0)

        def wait_rows():
            pltpu.make_async_copy(
                table_hbm.at[pl.ds(0, _R)], rows, sem_row).wait()

        def do_overflow(kv, posv, m):
            novf = jnp.sum(m.astype(jnp.int32))

            @pl.when(novf > 0)
            def _():
                ovf_key[pl.ds(0, 16)] = jnp.zeros((16,), jnp.int32)
                pf = plsc.cumsum(m.astype(jnp.int32))
                dst = jnp.where(m, pf - 1, 0)
                plsc.store_scatter(ovf_key, [dst], kv, mask=m)
                for b in range(2):
                    @pl.when(novf > 8 * b)
                    def _():
                        pltpu.async_copy(
                            table_hbm.at[ovf_key.at[pl.ds(8 * b, 8)]],
                            rows, sem_row)
                        wait_rows()
                        cb = jnp.minimum(novf - 8 * b, 8)

                        def issue(j, c):
                            srcm = m & (jnp.where(m, pf - 1, -1) == (8 * b + j))
                            pp = jnp.sum(jnp.where(srcm, posv, 0))
                            pltpu.async_copy(
                                rows.at[j], out_hbm.at[pp], sem_out)
                            return c
                        lax.fori_loop(0, cb, issue, 0)
                        drain_outs(cb)

        cursor = jnp.int32(0)
        for ch in range(n_idx_chunks):
            pltpu.sync_copy(
                idx_hbm.at[pl.ds(ch * chunk_len, chunk_len)], idxbuf)

            def inner(i, cur, ch=ch):
                kv = idxbuf[pl.ds(pl.multiple_of(i * 16, 16), 16)]
                m = (kv >= lo) & (kv < hi)
                posv = ch * chunk_len + i * 16 + lane
                pf = plsc.cumsum(m.astype(jnp.int32))
                dst = cur + pf - 1
                sel = m & (dst < _CAP_E)
                dstc = jnp.where(sel, dst, 0)
                plsc.store_scatter(e_pos, [dstc], posv, mask=sel)
                plsc.store_scatter(e_key, [dstc], kv, mask=sel)
                do_overflow(kv, posv, m & (dst >= _CAP_E))
                return cur + jnp.sum(m.astype(jnp.int32))

            cursor = lax.fori_loop(0, chunk_len // 16, inner, cursor)

        ec = jnp.minimum(cursor, _CAP_E)
        nv = (ec + 15) // 16

        def build(s, tot):
            v0 = lo + s * _R

            def rescan(j, c2):
                kk = e_key[pl.ds(pl.multiple_of(j * 16, 16), 16)]
                pp = e_pos[pl.ds(pl.multiple_of(j * 16, 16), 16)]
                m = ((j * 16 + lane) < ec) & (kk >= v0) & (kk < v0 + _R)
                pf = plsc.cumsum(m.astype(jnp.int32))
                dst = jnp.where(m, c2 + pf - 1, 0)
                plsc.store_scatter(g_pos, [dst], pp, mask=m)
                plsc.store_scatter(g_key, [dst], kk, mask=m)
                return c2 + jnp.sum(m.astype(jnp.int32))

            return lax.fori_loop(0, nv, rescan, tot)

        tot = lax.fori_loop(0, n_sub, build, jnp.int32(0))

        def entry(e, carry):
            cur_s, pending = carry
            vr = jnp.right_shift(e, 4)
            lj = jnp.bitwise_and(e, 15)
            kvv = g_key[pl.ds(pl.multiple_of(vr * 16, 16), 16)]
            pvv = g_pos[pl.ds(pl.multiple_of(vr * 16, 16), 16)]
            key = lane_at(kvv, lj)
            pos = lane_at(pvv, lj)
            s_e = jnp.right_shift(key - lo, 3)
            row = jnp.bitwise_and(key - lo, _R - 1)
            new_span = s_e != cur_s

            @pl.when(new_span)
            def _():
                drain_outs(pending)
                pltpu.async_copy(
                    table_hbm.at[pl.ds(lo + s_e * _R, _R)], rows, sem_row)
                wait_rows()

            pltpu.async_copy(rows.at[row], out_hbm.at[pos], sem_out)
            return (s_e, jnp.where(new_span, 1, pending + 1))

        cur_s, pending = lax.fori_loop(
            0, tot, entry, (jnp.int32(-1), jnp.int32(0)))
        drain_outs(pending)

    return body


def kernel(idx, table):
    b, t = idx.shape
    v, d = table.shape
    out = _build(b * t, v, d)(idx.reshape(-1).astype(jnp.int32), table)
    return out.reshape(b, t, d)
